# baseline (speedup 1.0000x reference)
import functools

import jax
import jax.numpy as jnp
from jax import lax
from jax.experimental import pallas as pl
from jax.experimental.pallas import tpu as pltpu
from jax.experimental.pallas import tpu_sc as plsc

EMBED = 32
BATCH = 16384

NUM_CORES = 2
NUM_SUBCORES = 16
NUM_WORKERS = NUM_CORES * NUM_SUBCORES
B_PER_W = BATCH // NUM_WORKERS
CHUNK = 128
N_CHUNKS = B_PER_W // CHUNK


@functools.partial(
    pl.kernel,
    mesh=plsc.VectorSubcoreMesh(core_axis_name="c", subcore_axis_name="s"),
    out_type=(
        jax.ShapeDtypeStruct((EMBED, BATCH), jnp.float32),
        jax.ShapeDtypeStruct((EMBED, BATCH), jnp.float32),
    ),
    scratch_types=[
        pltpu.VMEM((N_CHUNKS, CHUNK), jnp.int32),
        pltpu.VMEM((N_CHUNKS, CHUNK), jnp.int32),
        pltpu.VMEM((EMBED, B_PER_W), jnp.float32),
        pltpu.VMEM((EMBED, B_PER_W), jnp.float32),
        pltpu.SemaphoreType.DMA,
        pltpu.SemaphoreType.DMA,
    ],
    compiler_params=pltpu.CompilerParams(use_tc_tiling_on_sc=False),
)
def _bpr_gather(
    user_hbm,
    item_hbm,
    user_t_hbm,
    item_t_hbm,
    user_out_hbm,
    item_out_hbm,
    uidx_v,
    iidx_v,
    urows_v,
    irows_v,
    usem,
    isem,
):
    wid = lax.axis_index("s") * NUM_CORES + lax.axis_index("c")
    base = wid * B_PER_W

    for c in range(N_CHUNKS):
        pltpu.sync_copy(user_hbm.at[pl.ds(base + c * CHUNK, CHUNK)], uidx_v.at[c])
        pltpu.sync_copy(item_hbm.at[pl.ds(base + c * CHUNK, CHUNK)], iidx_v.at[c])

    def plane_body(f, _):
        copies = []
        for c in range(N_CHUNKS):
            sl = pl.ds(c * CHUNK, CHUNK)
            copies.append(
                pltpu.async_copy(
                    user_t_hbm.at[f].at[uidx_v.at[c]], urows_v.at[f, sl], usem
                )
            )
            copies.append(
                pltpu.async_copy(
                    item_t_hbm.at[f].at[iidx_v.at[c]], irows_v.at[f, sl], isem
                )
            )
        for cp in copies:
            cp.wait()
        return 0

    lax.fori_loop(0, EMBED, plane_body, 0)

    pltpu.sync_copy(urows_v, user_out_hbm.at[:, pl.ds(base, B_PER_W)])
    pltpu.sync_copy(irows_v, item_out_hbm.at[:, pl.ds(base, B_PER_W)])


def kernel(user, item, user_table, item_table):
    u_t, i_t = _bpr_gather(user, item, user_table.T, item_table.T)
    return (u_t.T, i_t.T)


# speedup vs baseline: 5.6178x; 5.6178x over previous
import functools

import jax
import jax.numpy as jnp
from jax import lax
from jax.experimental import pallas as pl
from jax.experimental.pallas import tpu as pltpu
from jax.experimental.pallas import tpu_sc as plsc

EMBED = 32
BATCH = 16384

NUM_CORES = 2
NUM_SUBCORES = 16
NUM_WORKERS = NUM_CORES * NUM_SUBCORES
B_PER_W = BATCH // NUM_WORKERS
CHUNK = 128
N_CHUNKS = B_PER_W // CHUNK


@functools.partial(
    pl.kernel,
    mesh=plsc.VectorSubcoreMesh(core_axis_name="c", subcore_axis_name="s"),
    out_type=jax.ShapeDtypeStruct((BATCH, EMBED), jnp.float32),
    scratch_types=[
        pltpu.VMEM((B_PER_W,), jnp.int32),
        pltpu.VMEM((B_PER_W, EMBED), jnp.float32),
        pltpu.SemaphoreType.DMA,
    ],
    compiler_params=pltpu.CompilerParams(use_tc_tiling_on_sc=False),
)
def _gather_one(idx_hbm, table_hbm, out_hbm, idx_v, rows_v, sem):
    wid = lax.axis_index("s") * NUM_CORES + lax.axis_index("c")
    base = wid * B_PER_W

    pltpu.sync_copy(idx_hbm.at[pl.ds(base, B_PER_W)], idx_v)

    copies = []
    for j in range(N_CHUNKS):
        sl = pl.ds(j * CHUNK, CHUNK)
        copies.append(
            pltpu.async_copy(table_hbm.at[idx_v.at[sl]], rows_v.at[sl], sem)
        )
    for c in copies:
        c.wait()
    pltpu.sync_copy(rows_v, out_hbm.at[pl.ds(base, B_PER_W)])


def kernel(user, item, user_table, item_table):
    return (_gather_one(user, user_table), _gather_one(item, item_table))
